# Initial kernel scaffold; baseline (speedup 1.0000x reference)
import jax
import jax.numpy as jnp
from jax.experimental import pallas as pl


def kernel(x, pos, edge_index, batch, sigma, We1, We2, Wh, Wx):
    raise NotImplementedError("write your pallas kernel here")


# speedup vs baseline: 3.1170x; 3.1170x over previous
import functools

import jax
import jax.numpy as jnp
from jax import lax
from jax.experimental import pallas as pl
from jax.experimental.pallas import tpu as pltpu
from jax.experimental.pallas import tpu_sc as plsc

N = 10000
E = 320000
D = 128
B = 64
H = 128
SIGMA_DATA = 0.5

GE = 144
NW = 32
EPW = E // NW
CH = 80
NCH = EPW // CH
RPT = N // 16
ZCH = 125
EB = 2000
L = 16


NB = 2000
NNB = N // NB


def _precompute_body(x_ref, pos_ref, sig_ref, wsrc_ref, wdst_ref, wns_ref,
                     wnd_ref, p_ref, q_ref):
    sig = sig_ref[...]
    c_in = 1.0 / jnp.sqrt(SIGMA_DATA ** 2 + sig * sig)
    c_noise = jnp.log(sig) / 4.0
    xc = c_in * x_ref[...]
    posc = c_in * pos_ref[...]
    zpad = jnp.zeros((NB, GE - 131), jnp.float32)
    p = jnp.dot(xc, wsrc_ref[...], preferred_element_type=jnp.float32)
    p = p + c_noise * wns_ref[...]
    q = jnp.dot(xc, wdst_ref[...], preferred_element_type=jnp.float32)
    q = q + c_noise * wnd_ref[...]
    p_ref[...] = jnp.concatenate([p, posc, zpad], axis=1)
    q_ref[...] = jnp.concatenate(
        [q, jnp.zeros((NB, 4), jnp.float32), posc,
         jnp.zeros((NB, GE - 135), jnp.float32)], axis=1)


def _precompute(x, pos, sig2, wsrc, wdst, wns, wnd):
    nspec = lambda w: pl.BlockSpec((NB, w), lambda i: (i, 0))
    wspec = lambda a, b: pl.BlockSpec((a, b), lambda i: (0, 0))
    return pl.pallas_call(
        _precompute_body,
        grid=(NNB,),
        in_specs=[nspec(D), nspec(3), nspec(1), wspec(D, H), wspec(D, H),
                  wspec(1, H), wspec(1, H)],
        out_specs=(nspec(GE), nspec(GE)),
        out_shape=(jax.ShapeDtypeStruct((N, GE), jnp.float32),
                   jax.ShapeDtypeStruct((N, GE), jnp.float32)),
    )(x, pos, sig2, wsrc, wdst, wns, wnd)


def _gather_body(p_hbm, q_hbm, src_hbm, dst_hbm, g_hbm, sidx, didx, bufa, bufb):
    wid = lax.axis_index("s") * 2 + lax.axis_index("c")
    base = wid * EPW

    def chunk(j, c):
        off = base + j * CH
        pltpu.sync_copy(src_hbm.at[pl.ds(off, CH)], sidx)
        pltpu.sync_copy(dst_hbm.at[pl.ds(off, CH)], didx)
        pltpu.sync_copy(p_hbm.at[sidx], bufa)
        pltpu.sync_copy(q_hbm.at[didx], bufb)

        def row(r, c2):
            for g in range(GE // L):
                sl = pl.ds(g * L, L)
                bufa[r, sl] = bufa[r, sl] + bufb[r, sl]
            return c2

        lax.fori_loop(0, CH, row, 0)
        pltpu.sync_copy(bufa, g_hbm.at[pl.ds(off, CH)])
        return c

    lax.fori_loop(0, NCH, chunk, 0)


_gather = functools.partial(
    pl.kernel,
    out_type=jax.ShapeDtypeStruct((E, GE), jnp.float32),
    mesh=plsc.VectorSubcoreMesh(core_axis_name="c", subcore_axis_name="s"),
    compiler_params=pltpu.CompilerParams(use_tc_tiling_on_sc=False),
    scratch_types=[
        pltpu.VMEM((CH,), jnp.int32),
        pltpu.VMEM((CH,), jnp.int32),
        pltpu.VMEM((CH, GE), jnp.float32),
        pltpu.VMEM((CH, GE), jnp.float32),
    ],
)(_gather_body)


def _edge_mlp_body(w2_ref, wx_ref, w259_ref, g_ref, out_ref):
    g = g_ref[...]
    rel = g[:, 128:131] - g[:, 132:135]
    d2 = jnp.sum(rel * rel, axis=1, keepdims=True)
    m0 = g[:, :128] + d2 * w259_ref[...]
    m1 = jax.nn.silu(m0)
    m2 = jax.nn.silu(jnp.dot(m1, w2_ref[...], preferred_element_type=jnp.float32))
    coef = jnp.tanh(jnp.dot(m2, wx_ref[...], preferred_element_type=jnp.float32))
    out_ref[...] = jnp.concatenate(
        [m2, rel * coef, jnp.ones((EB, 1), jnp.float32),
         jnp.zeros((EB, GE - 132), jnp.float32)], axis=1)


def _edge_mlp(w2, wx, w259, gmat):
    return pl.pallas_call(
        _edge_mlp_body,
        grid=(E // EB,),
        in_specs=[
            pl.BlockSpec((H, H), lambda i: (0, 0)),
            pl.BlockSpec((H, 1), lambda i: (0, 0)),
            pl.BlockSpec((1, H), lambda i: (0, 0)),
            pl.BlockSpec((EB, GE), lambda i: (i, 0)),
        ],
        out_specs=pl.BlockSpec((EB, GE), lambda i: (i, 0)),
        out_shape=jax.ShapeDtypeStruct((E, GE), jnp.float32),
    )(w2, wx, w259, gmat)


def _scatter_body(m_hbm, dst_hbm, o_hbm, ibuf, mbuf, zbuf, acc):
    cid = lax.axis_index("c")
    sid = lax.axis_index("s")

    def zrow(r, c):
        for g in range(GE // L):
            zbuf[r, pl.ds(g * L, L)] = jnp.zeros((L,), jnp.float32)
        return c

    lax.fori_loop(0, ZCH, zrow, 0)

    def zchunk(k, c):
        pltpu.sync_copy(zbuf, acc.at[pl.ds(sid * RPT + k * ZCH, ZCH)])
        return c

    lax.fori_loop(0, RPT // ZCH, zchunk, 0)
    plsc.subcore_barrier()

    base = (cid * 16 + sid) * EPW

    def chunk(j, c):
        off = base + j * CH
        pltpu.sync_copy(dst_hbm.at[pl.ds(off, CH)], ibuf)
        pltpu.sync_copy(m_hbm.at[pl.ds(off, CH)], mbuf)
        pltpu.sync_copy(mbuf, acc.at[ibuf], add=True)
        return c

    lax.fori_loop(0, NCH, chunk, 0)
    plsc.subcore_barrier()

    def ochunk(k, c):
        r0 = sid * RPT + k * ZCH
        pltpu.sync_copy(acc.at[pl.ds(r0, ZCH)], o_hbm.at[cid, pl.ds(r0, ZCH)])
        return c

    lax.fori_loop(0, RPT // ZCH, ochunk, 0)


_scatter = functools.partial(
    pl.kernel,
    out_type=jax.ShapeDtypeStruct((2, N, GE), jnp.float32),
    mesh=plsc.VectorSubcoreMesh(core_axis_name="c", subcore_axis_name="s"),
    compiler_params=pltpu.CompilerParams(use_tc_tiling_on_sc=False),
    scratch_types=[
        pltpu.VMEM((CH,), jnp.int32),
        pltpu.VMEM((CH, GE), jnp.float32),
        pltpu.VMEM((ZCH, GE), jnp.float32),
        pltpu.VMEM_SHARED((N, GE), jnp.float32),
    ],
)(_scatter_body)


def _node_q(part_ref):
    p0 = part_ref[0]
    p1 = part_ref[1]
    upd = p0[:, 128:131] + p1[:, 128:131]
    cnt = p0[:, 131:132] + p1[:, 131:132]
    cntg = jnp.where(cnt == 0.0, 1.0, cnt)
    return upd / cntg


def _graph_mean_body(bat_ref, part_ref, mc_ref):
    q = _node_q(part_ref)
    onehot = (bat_ref[...] == lax.broadcasted_iota(jnp.int32, (1, B), 1)
              ).astype(jnp.float32)
    rhs = jnp.concatenate([q, jnp.ones((NB, 1), jnp.float32)], axis=1)
    contrib = lax.dot_general(onehot, rhs, (((0,), (0,)), ((), ())),
                              preferred_element_type=jnp.float32)

    @pl.when(pl.program_id(0) == 0)
    def _():
        mc_ref[...] = jnp.zeros_like(mc_ref)

    mc_ref[...] += contrib


def _graph_mean(bat2, partials):
    return pl.pallas_call(
        _graph_mean_body,
        grid=(NNB,),
        in_specs=[
            pl.BlockSpec((NB, 1), lambda i: (i, 0)),
            pl.BlockSpec((2, NB, GE), lambda i: (0, i, 0)),
        ],
        out_specs=pl.BlockSpec((B, 4), lambda i: (0, 0)),
        out_shape=jax.ShapeDtypeStruct((B, 4), jnp.float32),
    )(bat2, partials)


def _finalize_body(x_ref, pos_ref, sig_ref, bat_ref, part_ref, mc_ref,
                   whx_ref, whn_ref, wha_ref, dx_ref, dpos_ref):
    sig = sig_ref[...]
    s2 = sig * sig
    c_skip = SIGMA_DATA ** 2 / (s2 + SIGMA_DATA ** 2)
    c_out = sig * SIGMA_DATA / jnp.sqrt(s2 + SIGMA_DATA ** 2)
    c_in = 1.0 / jnp.sqrt(SIGMA_DATA ** 2 + s2)
    c_noise = jnp.log(sig) / 4.0

    p0 = part_ref[0]
    p1 = part_ref[1]
    agg = p0[:, :128] + p1[:, :128]

    xc = c_in * x_ref[...]
    dx = jnp.dot(xc, whx_ref[...], preferred_element_type=jnp.float32)
    dx = dx + c_noise * whn_ref[...]
    dx = dx + jnp.dot(agg, wha_ref[...], preferred_element_type=jnp.float32)
    f_x = xc - dx

    q = _node_q(part_ref)
    mc = mc_ref[...]
    means = mc[:, :3] / jnp.where(mc[:, 3:] == 0.0, 1.0, mc[:, 3:])
    onehot = (bat_ref[...] == lax.broadcasted_iota(jnp.int32, (1, B), 1)
              ).astype(jnp.float32)
    centered = q - jnp.dot(onehot, means, preferred_element_type=jnp.float32)

    pos_in = c_in * pos_ref[...]
    f_pos = pos_in + centered
    dx_ref[...] = c_skip * x_ref[...] + c_out * f_x
    dpos_ref[...] = c_skip * pos_ref[...] + c_out * f_pos


def _finalize(x, pos, sig2, bat2, partials, mc, whx, whn, wha):
    nspec = lambda w: pl.BlockSpec((NB, w), lambda i: (i, 0))
    wspec = lambda a, b: pl.BlockSpec((a, b), lambda i: (0, 0))
    return pl.pallas_call(
        _finalize_body,
        grid=(NNB,),
        in_specs=[
            nspec(D), nspec(3), nspec(1), nspec(1),
            pl.BlockSpec((2, NB, GE), lambda i: (0, i, 0)),
            wspec(B, 4), wspec(D, H), wspec(1, H), wspec(D, H),
        ],
        out_specs=(nspec(D), nspec(3)),
        out_shape=(jax.ShapeDtypeStruct((N, D), jnp.float32),
                   jax.ShapeDtypeStruct((N, 3), jnp.float32)),
    )(x, pos, sig2, bat2, partials, mc, whx, whn, wha)


def kernel(x, pos, edge_index, batch, sigma, We1, We2, Wh, Wx):
    sig2 = sigma.reshape(N, 1)
    bat2 = batch.reshape(N, 1)
    src = edge_index[0]
    dst = edge_index[1]
    wsrc = We1[0:128]
    wns = We1[128:129]
    wdst = We1[129:257]
    wnd = We1[257:258]
    w259 = We1[258:259]
    whx = Wh[0:128]
    whn = Wh[128:129]
    wha = Wh[129:257]

    p_ext, q_ext = _precompute(x, pos, sig2, wsrc, wdst, wns, wnd)
    gmat = _gather(p_ext, q_ext, src, dst)
    mmat = _edge_mlp(We2, Wx, w259, gmat)
    partials = _scatter(mmat, dst)
    mc = _graph_mean(bat2, partials)
    return _finalize(x, pos, sig2, bat2, partials, mc, whx, whn, wha)


# speedup vs baseline: 3.5675x; 1.1445x over previous
import functools

import jax
import jax.numpy as jnp
from jax import lax
from jax.experimental import pallas as pl
from jax.experimental.pallas import tpu as pltpu
from jax.experimental.pallas import tpu_sc as plsc

N = 10000
E = 320000
D = 128
B = 64
H = 128
SIGMA_DATA = 0.5

GE = 144
NW = 32
EPW = E // NW
CH = 80
NCH = EPW // CH
RPT = N // 16
ZCH = 25
EB = 2000
L = 16


NB = 2000
NNB = N // NB


def _precompute_body(x_ref, pos_ref, sig_ref, wsrc_ref, wdst_ref, wns_ref,
                     wnd_ref, p_ref, q_ref):
    sig = sig_ref[...]
    c_in = 1.0 / jnp.sqrt(SIGMA_DATA ** 2 + sig * sig)
    c_noise = jnp.log(sig) / 4.0
    xc = c_in * x_ref[...]
    posc = c_in * pos_ref[...]
    zpad = jnp.zeros((NB, GE - 131), jnp.float32)
    p = jnp.dot(xc, wsrc_ref[...], preferred_element_type=jnp.float32)
    p = p + c_noise * wns_ref[...]
    q = jnp.dot(xc, wdst_ref[...], preferred_element_type=jnp.float32)
    q = q + c_noise * wnd_ref[...]
    p_ref[...] = jnp.concatenate([p, posc, zpad], axis=1)
    q_ref[...] = jnp.concatenate(
        [q, jnp.zeros((NB, 4), jnp.float32), posc,
         jnp.zeros((NB, GE - 135), jnp.float32)], axis=1)


def _precompute(x, pos, sig2, wsrc, wdst, wns, wnd):
    nspec = lambda w: pl.BlockSpec((NB, w), lambda i: (i, 0))
    wspec = lambda a, b: pl.BlockSpec((a, b), lambda i: (0, 0))
    return pl.pallas_call(
        _precompute_body,
        grid=(NNB,),
        in_specs=[nspec(D), nspec(3), nspec(1), wspec(D, H), wspec(D, H),
                  wspec(1, H), wspec(1, H)],
        out_specs=(nspec(GE), nspec(GE)),
        out_shape=(jax.ShapeDtypeStruct((N, GE), jnp.float32),
                   jax.ShapeDtypeStruct((N, GE), jnp.float32)),
    )(x, pos, sig2, wsrc, wdst, wns, wnd)


def _gather_body(p_hbm, q_hbm, src3_hbm, dst3_hbm, g_hbm,
                 sall, dall, ba0, ba1, bb0, bb1, bc0, bc1,
                 sa0, sa1, sb0, sb1, sw0, sw1):
    wid = lax.axis_index("s") * 2 + lax.axis_index("c")
    base = wid * EPW
    pltpu.sync_copy(src3_hbm.at[wid], sall)
    pltpu.sync_copy(dst3_hbm.at[wid], dall)
    ba = (ba0, ba1)
    bb = (bb0, bb1)
    bc = (bc0, bc1)
    sa = (sa0, sa1)
    sb = (sb0, sb1)
    sw = (sw0, sw1)

    def start(j, s):
        pltpu.async_copy(p_hbm.at[sall.at[j]], ba[s], sa[s])
        pltpu.async_copy(q_hbm.at[dall.at[j]], bb[s], sb[s])

    start(0, 0)
    start(1, 1)

    def half(i, j, s):
        pltpu.make_async_copy(p_hbm.at[sall.at[j]], ba[s], sa[s]).wait()
        pltpu.make_async_copy(q_hbm.at[dall.at[j]], bb[s], sb[s]).wait()

        @pl.when(i > 0)
        def _():
            pltpu.make_async_copy(
                bc[s], g_hbm.at[pl.ds(base + (j - 2) * CH, CH)], sw[s]).wait()

        def row(r, c2):
            for g in range(GE // L):
                sl = pl.ds(g * L, L)
                bc[s][r, sl] = ba[s][r, sl] + bb[s][r, sl]
            return c2

        lax.fori_loop(0, CH, row, 0, unroll=2)

        @pl.when(j + 2 < NCH)
        def _():
            start(j + 2, s)

        pltpu.async_copy(bc[s], g_hbm.at[pl.ds(base + j * CH, CH)], sw[s])

    def pair(i, c):
        half(i, 2 * i, 0)
        half(i, 2 * i + 1, 1)
        return c

    lax.fori_loop(0, NCH // 2, pair, 0)
    half(NCH // 2, NCH - 1, 0)
    pltpu.make_async_copy(
        bc[1], g_hbm.at[pl.ds(base + (NCH - 2) * CH, CH)], sw[1]).wait()
    pltpu.make_async_copy(
        bc[0], g_hbm.at[pl.ds(base + (NCH - 1) * CH, CH)], sw[0]).wait()


_gather = functools.partial(
    pl.kernel,
    out_type=jax.ShapeDtypeStruct((E, GE), jnp.float32),
    mesh=plsc.VectorSubcoreMesh(core_axis_name="c", subcore_axis_name="s"),
    compiler_params=pltpu.CompilerParams(use_tc_tiling_on_sc=False),
    scratch_types=[
        pltpu.VMEM((NCH, CH), jnp.int32),
        pltpu.VMEM((NCH, CH), jnp.int32),
        pltpu.VMEM((CH, GE), jnp.float32),
        pltpu.VMEM((CH, GE), jnp.float32),
        pltpu.VMEM((CH, GE), jnp.float32),
        pltpu.VMEM((CH, GE), jnp.float32),
        pltpu.VMEM((CH, GE), jnp.float32),
        pltpu.VMEM((CH, GE), jnp.float32),
        pltpu.SemaphoreType.DMA,
        pltpu.SemaphoreType.DMA,
        pltpu.SemaphoreType.DMA,
        pltpu.SemaphoreType.DMA,
        pltpu.SemaphoreType.DMA,
        pltpu.SemaphoreType.DMA,
    ],
)(_gather_body)


def _edge_mlp_body(w2_ref, wx_ref, w259_ref, g_ref, out_ref):
    g = g_ref[...]
    rel = g[:, 128:131] - g[:, 132:135]
    d2 = jnp.sum(rel * rel, axis=1, keepdims=True)
    m0 = g[:, :128] + d2 * w259_ref[...]
    m1 = jax.nn.silu(m0)
    m2 = jax.nn.silu(jnp.dot(m1, w2_ref[...], preferred_element_type=jnp.float32))
    coef = jnp.tanh(jnp.dot(m2, wx_ref[...], preferred_element_type=jnp.float32))
    out_ref[...] = jnp.concatenate(
        [m2, rel * coef, jnp.ones((EB, 1), jnp.float32),
         jnp.zeros((EB, GE - 132), jnp.float32)], axis=1)


def _edge_mlp(w2, wx, w259, gmat):
    return pl.pallas_call(
        _edge_mlp_body,
        grid=(E // EB,),
        in_specs=[
            pl.BlockSpec((H, H), lambda i: (0, 0)),
            pl.BlockSpec((H, 1), lambda i: (0, 0)),
            pl.BlockSpec((1, H), lambda i: (0, 0)),
            pl.BlockSpec((EB, GE), lambda i: (i, 0)),
        ],
        out_specs=pl.BlockSpec((EB, GE), lambda i: (i, 0)),
        out_shape=jax.ShapeDtypeStruct((E, GE), jnp.float32),
    )(w2, wx, w259, gmat)


def _scatter_body(m_hbm, dst3_hbm, o_hbm, dall, mb0, mb1, zbuf, acc, sm0, sm1):
    cid = lax.axis_index("c")
    sid = lax.axis_index("s")
    wid = cid * 16 + sid
    base = wid * EPW
    mb = (mb0, mb1)
    sm = (sm0, sm1)

    def zrow(r, c):
        for g in range(GE // L):
            zbuf[r, pl.ds(g * L, L)] = jnp.zeros((L,), jnp.float32)
        return c

    lax.fori_loop(0, ZCH, zrow, 0)

    def zchunk(k, c):
        pltpu.sync_copy(zbuf, acc.at[pl.ds(sid * RPT + k * ZCH, ZCH)])
        return c

    lax.fori_loop(0, RPT // ZCH, zchunk, 0)
    pltpu.sync_copy(dst3_hbm.at[wid], dall)
    plsc.subcore_barrier()

    def startm(j, s):
        pltpu.async_copy(m_hbm.at[pl.ds(base + j * CH, CH)], mb[s], sm[s])

    startm(0, 0)
    startm(1, 1)

    def half(j, s):
        pltpu.make_async_copy(
            m_hbm.at[pl.ds(base + j * CH, CH)], mb[s], sm[s]).wait()
        pltpu.sync_copy(mb[s], acc.at[dall.at[j]], add=True)

        @pl.when(j + 2 < NCH)
        def _():
            startm(j + 2, s)

    def pair(i, c):
        half(2 * i, 0)
        half(2 * i + 1, 1)
        return c

    lax.fori_loop(0, NCH // 2, pair, 0)
    half(NCH - 1, 0)
    plsc.subcore_barrier()

    def ochunk(k, c):
        r0 = sid * RPT + k * ZCH
        pltpu.sync_copy(acc.at[pl.ds(r0, ZCH)], o_hbm.at[cid, pl.ds(r0, ZCH)])
        return c

    lax.fori_loop(0, RPT // ZCH, ochunk, 0)


_scatter = functools.partial(
    pl.kernel,
    out_type=jax.ShapeDtypeStruct((2, N, GE), jnp.float32),
    mesh=plsc.VectorSubcoreMesh(core_axis_name="c", subcore_axis_name="s"),
    compiler_params=pltpu.CompilerParams(use_tc_tiling_on_sc=False),
    scratch_types=[
        pltpu.VMEM((NCH, CH), jnp.int32),
        pltpu.VMEM((CH, GE), jnp.float32),
        pltpu.VMEM((CH, GE), jnp.float32),
        pltpu.VMEM((ZCH, GE), jnp.float32),
        pltpu.VMEM_SHARED((N, GE), jnp.float32),
        pltpu.SemaphoreType.DMA,
        pltpu.SemaphoreType.DMA,
    ],
)(_scatter_body)


def _node_q(part_ref):
    p0 = part_ref[0]
    p1 = part_ref[1]
    upd = p0[:, 128:131] + p1[:, 128:131]
    cnt = p0[:, 131:132] + p1[:, 131:132]
    cntg = jnp.where(cnt == 0.0, 1.0, cnt)
    return upd / cntg


def _graph_mean_body(bat_ref, part_ref, mc_ref):
    q = _node_q(part_ref)
    onehot = (bat_ref[...] == lax.broadcasted_iota(jnp.int32, (1, B), 1)
              ).astype(jnp.float32)
    rhs = jnp.concatenate([q, jnp.ones((NB, 1), jnp.float32)], axis=1)
    contrib = lax.dot_general(onehot, rhs, (((0,), (0,)), ((), ())),
                              preferred_element_type=jnp.float32)

    @pl.when(pl.program_id(0) == 0)
    def _():
        mc_ref[...] = jnp.zeros_like(mc_ref)

    mc_ref[...] += contrib


def _graph_mean(bat2, partials):
    return pl.pallas_call(
        _graph_mean_body,
        grid=(NNB,),
        in_specs=[
            pl.BlockSpec((NB, 1), lambda i: (i, 0)),
            pl.BlockSpec((2, NB, GE), lambda i: (0, i, 0)),
        ],
        out_specs=pl.BlockSpec((B, 4), lambda i: (0, 0)),
        out_shape=jax.ShapeDtypeStruct((B, 4), jnp.float32),
    )(bat2, partials)


def _finalize_body(x_ref, pos_ref, sig_ref, bat_ref, part_ref, mc_ref,
                   whx_ref, whn_ref, wha_ref, dx_ref, dpos_ref):
    sig = sig_ref[...]
    s2 = sig * sig
    c_skip = SIGMA_DATA ** 2 / (s2 + SIGMA_DATA ** 2)
    c_out = sig * SIGMA_DATA / jnp.sqrt(s2 + SIGMA_DATA ** 2)
    c_in = 1.0 / jnp.sqrt(SIGMA_DATA ** 2 + s2)
    c_noise = jnp.log(sig) / 4.0

    p0 = part_ref[0]
    p1 = part_ref[1]
    agg = p0[:, :128] + p1[:, :128]

    xc = c_in * x_ref[...]
    dx = jnp.dot(xc, whx_ref[...], preferred_element_type=jnp.float32)
    dx = dx + c_noise * whn_ref[...]
    dx = dx + jnp.dot(agg, wha_ref[...], preferred_element_type=jnp.float32)
    f_x = xc - dx

    q = _node_q(part_ref)
    mc = mc_ref[...]
    means = mc[:, :3] / jnp.where(mc[:, 3:] == 0.0, 1.0, mc[:, 3:])
    onehot = (bat_ref[...] == lax.broadcasted_iota(jnp.int32, (1, B), 1)
              ).astype(jnp.float32)
    centered = q - jnp.dot(onehot, means, preferred_element_type=jnp.float32)

    pos_in = c_in * pos_ref[...]
    f_pos = pos_in + centered
    dx_ref[...] = c_skip * x_ref[...] + c_out * f_x
    dpos_ref[...] = c_skip * pos_ref[...] + c_out * f_pos


def _finalize(x, pos, sig2, bat2, partials, mc, whx, whn, wha):
    nspec = lambda w: pl.BlockSpec((NB, w), lambda i: (i, 0))
    wspec = lambda a, b: pl.BlockSpec((a, b), lambda i: (0, 0))
    return pl.pallas_call(
        _finalize_body,
        grid=(NNB,),
        in_specs=[
            nspec(D), nspec(3), nspec(1), nspec(1),
            pl.BlockSpec((2, NB, GE), lambda i: (0, i, 0)),
            wspec(B, 4), wspec(D, H), wspec(1, H), wspec(D, H),
        ],
        out_specs=(nspec(D), nspec(3)),
        out_shape=(jax.ShapeDtypeStruct((N, D), jnp.float32),
                   jax.ShapeDtypeStruct((N, 3), jnp.float32)),
    )(x, pos, sig2, bat2, partials, mc, whx, whn, wha)


def kernel(x, pos, edge_index, batch, sigma, We1, We2, Wh, Wx):
    sig2 = sigma.reshape(N, 1)
    bat2 = batch.reshape(N, 1)
    src3 = edge_index[0].reshape(NW, NCH, CH)
    dst3 = edge_index[1].reshape(NW, NCH, CH)
    wsrc = We1[0:128]
    wns = We1[128:129]
    wdst = We1[129:257]
    wnd = We1[257:258]
    w259 = We1[258:259]
    whx = Wh[0:128]
    whn = Wh[128:129]
    wha = Wh[129:257]

    p_ext, q_ext = _precompute(x, pos, sig2, wsrc, wdst, wns, wnd)
    gmat = _gather(p_ext, q_ext, src3, dst3)
    mmat = _edge_mlp(We2, Wx, w259, gmat)
    partials = _scatter(mmat, dst3)
    mc = _graph_mean(bat2, partials)
    return _finalize(x, pos, sig2, bat2, partials, mc, whx, whn, wha)
